# baseline (speedup 1.0000x reference)
import functools

import jax
import jax.numpy as jnp
from jax import lax
from jax.experimental import pallas as pl
from jax.experimental.pallas import tpu as pltpu
from jax.experimental.pallas import tpu_sc as plsc

_VOCAB = 1000
_NC = 2
_NS = 16
_NW = _NC * _NS
_LANES = 16
_BW = 32


def _prep_body(t_ref, i_ref, g_ref, o_ref, it_ref, gt_ref):
    x = t_ref[...]
    m = jnp.max(x, axis=1)
    s = jnp.sum(jnp.exp(x - m[:, None]), axis=1)
    o_ref[...] = m + jnp.log(s)
    it_ref[...] = i_ref[...].T
    gt_ref[...] = g_ref[...].T


def _tc_prep(table, idx, targets):
    b, l = idx.shape
    return pl.pallas_call(
        _prep_body,
        out_shape=(
            jax.ShapeDtypeStruct((table.shape[0],), jnp.float32),
            jax.ShapeDtypeStruct((l, b), jnp.int32),
            jax.ShapeDtypeStruct((l, b), jnp.int32),
        ),
    )(table, idx, targets)


def _make_sc_kernel(n_b, n_l):
    assert n_b == _NW * _BW
    vt_n = _VOCAB // 8
    bt_n = n_b // 128
    mesh = plsc.VectorSubcoreMesh(core_axis_name="c", subcore_axis_name="s")

    @functools.partial(
        pl.kernel,
        out_type=(
            jax.ShapeDtypeStruct((n_l, vt_n, bt_n, 8, 128), jnp.float32),
            jax.ShapeDtypeStruct((_NW, _LANES), jnp.float32),
        ),
        mesh=mesh,
        compiler_params=pltpu.CompilerParams(
            use_tc_tiling_on_sc=False, needs_layout_passes=False),
        scratch_types=[
            pltpu.VMEM((_BW,), jnp.int32),
            pltpu.VMEM((_BW,), jnp.int32),
            pltpu.VMEM((_BW,), jnp.int32),
            pltpu.VMEM((_BW,), jnp.int32),
            pltpu.VMEM((_VOCAB,), jnp.float32),
            pltpu.VMEM((_BW, _VOCAB), jnp.float32),
            pltpu.VMEM((_BW, _VOCAB), jnp.float32),
            pltpu.VMEM((vt_n, 8, _BW), jnp.float32),
            pltpu.VMEM((vt_n, 8, _BW), jnp.float32),
            pltpu.VMEM((_LANES,), jnp.float32),
            pltpu.SemaphoreType.DMA,
            pltpu.SemaphoreType.DMA,
            pltpu.SemaphoreType.DMA,
            pltpu.SemaphoreType.DMA,
            pltpu.SemaphoreType.DMA,
            pltpu.SemaphoreType.DMA,
        ],
    )
    def sc_kernel(table_hbm, idxt_hbm, tgtt_hbm, lse_hbm, y5_hbm, part_hbm,
                  idxu_a, idxu_b, tgtu_a, tgtu_b, lse_v, gbuf_a, gbuf_b,
                  obuf_a, obuf_b, acc_v, gsem_a, gsem_b, ssem_a, ssem_b,
                  psem_a, psem_b):
        wid = lax.axis_index("s") * _NC + lax.axis_index("c")
        bt = wid // 4
        lane0 = (wid % 4) * _BW
        b0 = wid * _BW
        pltpu.sync_copy(lse_hbm, lse_v)
        acc_v[...] = jnp.zeros((_LANES,), jnp.float32)
        lane = lax.broadcasted_iota(jnp.int32, (_LANES,), 0)

        def prep(l, idxu, tgtu):
            pltpu.sync_copy(idxt_hbm.at[l, pl.ds(b0, _BW)], idxu)
            pltpu.sync_copy(tgtt_hbm.at[l, pl.ds(b0, _BW)], tgtu)

        def gstart(idxu, gbuf, gsem):
            pltpu.make_async_copy(table_hbm.at[idxu], gbuf, gsem).start()

        def gwait(idxu, gbuf, gsem):
            pltpu.make_async_copy(table_hbm.at[idxu], gbuf, gsem).wait()

        def swait(l, obuf, ssem):
            dst = y5_hbm.at[l, :, bt, :, pl.ds(lane0, _BW)]
            pltpu.make_async_copy(obuf, dst, ssem).wait()

        def unit(l, refill_l, idxu, tgtu, gbuf, obuf, gsem, ssem, psem,
                 first):
            gwait(idxu, gbuf, gsem)
            for j0 in range(0, _BW, _LANES):
                ig = idxu[pl.ds(j0, _LANES)]
                tg = tgtu[pl.ds(j0, _LANES)]
                picked = plsc.load_gather(gbuf, [lane + j0, tg])
                lseg = plsc.load_gather(lse_v, [ig])
                acc_v[...] = acc_v[...] + (lseg - picked)
            if refill_l is not None:
                pltpu.make_async_copy(
                    idxt_hbm.at[refill_l, pl.ds(b0, _BW)], idxu, psem).start()
                pltpu.make_async_copy(
                    tgtt_hbm.at[refill_l, pl.ds(b0, _BW)], tgtu, psem).start()
            if not first:
                swait(l, obuf, ssem)

            def tr_body(vt):
                lo, hi = [], []
                for vs in range(8):
                    vcol = jnp.full((_LANES,), 0, jnp.int32) + (vt * 8 + vs)
                    lo.append(plsc.load_gather(gbuf, [lane, vcol]))
                    hi.append(plsc.load_gather(gbuf, [lane + _LANES, vcol]))
                for vs in range(8):
                    obuf[vt, vs, pl.ds(0, _LANES)] = lo[vs]
                    obuf[vt, vs, pl.ds(_LANES, _LANES)] = hi[vs]

            plsc.parallel_loop(0, vt_n, step=1, unroll=2)(tr_body)
            if refill_l is not None:
                pltpu.make_async_copy(
                    idxt_hbm.at[0, pl.ds(b0, _BW)], idxu, psem).wait()
                pltpu.make_async_copy(
                    tgtt_hbm.at[0, pl.ds(b0, _BW)], tgtu, psem).wait()
                gstart(idxu, gbuf, gsem)
            dst = y5_hbm.at[l, :, bt, :, pl.ds(lane0, _BW)]
            pltpu.make_async_copy(obuf, dst, ssem).start()

        prep(0, idxu_a, tgtu_a)
        gstart(idxu_a, gbuf_a, gsem_a)
        prep(1, idxu_b, tgtu_b)
        gstart(idxu_b, gbuf_b, gsem_b)

        unit(0, 2, idxu_a, tgtu_a, gbuf_a, obuf_a, gsem_a, ssem_a, psem_a,
             True)
        unit(1, 3, idxu_b, tgtu_b, gbuf_b, obuf_b, gsem_b, ssem_b, psem_b,
             True)

        def pair_body(p, carry):
            l = 2 * p
            unit(l, l + 2, idxu_a, tgtu_a, gbuf_a, obuf_a, gsem_a, ssem_a,
                 psem_a, False)
            unit(l + 1, l + 3, idxu_b, tgtu_b, gbuf_b, obuf_b, gsem_b,
                 ssem_b, psem_b, False)
            return carry

        lax.fori_loop(1, n_l // 2 - 1, pair_body, 0)

        unit(n_l - 2, None, idxu_a, tgtu_a, gbuf_a, obuf_a, gsem_a, ssem_a,
             psem_a, False)
        unit(n_l - 1, None, idxu_b, tgtu_b, gbuf_b, obuf_b, gsem_b, ssem_b,
             psem_b, False)
        swait(n_l - 2, obuf_a, ssem_a)
        swait(n_l - 1, obuf_b, ssem_b)
        pltpu.sync_copy(acc_v, part_hbm.at[wid])

    return sc_kernel


def kernel(idx, targets, table):
    b, l = idx.shape
    n_tok = b * l
    lse, idxt, tgtt = _tc_prep(
        table, idx.astype(jnp.int32), targets.astype(jnp.int32))
    y5, partials = _make_sc_kernel(b, l)(table, idxt, tgtt, lse)
    logits = jnp.transpose(y5, (2, 4, 0, 1, 3)).reshape(b, l, _VOCAB)
    loss = jnp.sum(partials) / n_tok
    return (logits, loss)


# speedup vs baseline: 1.0475x; 1.0475x over previous
import functools

import jax
import jax.numpy as jnp
from jax import lax
from jax.experimental import pallas as pl
from jax.experimental.pallas import tpu as pltpu
from jax.experimental.pallas import tpu_sc as plsc

_VOCAB = 1000
_NC = 2
_NS = 16
_NW = _NC * _NS
_LANES = 16
_BW = 32


def _lse_body(t_ref, o_ref):
    x = t_ref[...]
    m = jnp.max(x, axis=1)
    s = jnp.sum(jnp.exp(x - m[:, None]), axis=1)
    o_ref[...] = m + jnp.log(s)


def _row_lse(table):
    return pl.pallas_call(
        _lse_body,
        out_shape=jax.ShapeDtypeStruct((table.shape[0],), jnp.float32),
    )(table)


def _make_sc_kernel(n_b, n_l):
    assert n_b == _NW * _BW
    vt_n = _VOCAB // 8
    bt_n = n_b // 128
    mesh = plsc.VectorSubcoreMesh(core_axis_name="c", subcore_axis_name="s")

    @functools.partial(
        pl.kernel,
        out_type=(
            jax.ShapeDtypeStruct((n_l, vt_n, bt_n, 8, 128), jnp.float32),
            jax.ShapeDtypeStruct((_NW, _LANES), jnp.float32),
        ),
        mesh=mesh,
        compiler_params=pltpu.CompilerParams(
            use_tc_tiling_on_sc=False, needs_layout_passes=False),
        scratch_types=[
            pltpu.VMEM((_BW,), jnp.int32),
            pltpu.VMEM((_BW,), jnp.int32),
            pltpu.VMEM((_BW,), jnp.int32),
            pltpu.VMEM((_BW,), jnp.int32),
            pltpu.VMEM((_VOCAB,), jnp.float32),
            pltpu.VMEM((_BW, _VOCAB), jnp.float32),
            pltpu.VMEM((_BW, _VOCAB), jnp.float32),
            pltpu.VMEM((vt_n, 8, _BW), jnp.float32),
            pltpu.VMEM((vt_n, 8, _BW), jnp.float32),
            pltpu.VMEM((_LANES,), jnp.float32),
            pltpu.SemaphoreType.DMA,
            pltpu.SemaphoreType.DMA,
            pltpu.SemaphoreType.DMA,
            pltpu.SemaphoreType.DMA,
            pltpu.SemaphoreType.DMA,
            pltpu.SemaphoreType.DMA,
        ],
    )
    def sc_kernel(table_hbm, idxt_hbm, tgtt_hbm, lse_hbm, y5_hbm, part_hbm,
                  idxu_a, idxu_b, tgtu_a, tgtu_b, lse_v, gbuf_a, gbuf_b,
                  obuf_a, obuf_b, acc_v, gsem_a, gsem_b, ssem_a, ssem_b,
                  psem_a, psem_b):
        wid = lax.axis_index("s") * _NC + lax.axis_index("c")
        bt = wid // 4
        lane0 = (wid % 4) * _BW
        b0 = wid * _BW
        pltpu.sync_copy(lse_hbm, lse_v)
        acc_v[...] = jnp.zeros((_LANES,), jnp.float32)
        lane = lax.broadcasted_iota(jnp.int32, (_LANES,), 0)

        def prep(l, idxu, tgtu):
            pltpu.sync_copy(idxt_hbm.at[l, pl.ds(b0, _BW)], idxu)
            pltpu.sync_copy(tgtt_hbm.at[l, pl.ds(b0, _BW)], tgtu)

        def gstart(idxu, gbuf, gsem):
            pltpu.make_async_copy(table_hbm.at[idxu], gbuf, gsem).start()

        def gwait(idxu, gbuf, gsem):
            pltpu.make_async_copy(table_hbm.at[idxu], gbuf, gsem).wait()

        def swait(l, obuf, ssem):
            dst = y5_hbm.at[l, :, bt, :, pl.ds(lane0, _BW)]
            pltpu.make_async_copy(obuf, dst, ssem).wait()

        def unit(l, refill_l, idxu, tgtu, gbuf, obuf, gsem, ssem, psem,
                 first):
            gwait(idxu, gbuf, gsem)
            for j0 in range(0, _BW, _LANES):
                ig = idxu[pl.ds(j0, _LANES)]
                tg = tgtu[pl.ds(j0, _LANES)]
                picked = plsc.load_gather(gbuf, [lane + j0, tg])
                lseg = plsc.load_gather(lse_v, [ig])
                acc_v[...] = acc_v[...] + (lseg - picked)
            if refill_l is not None:
                pltpu.make_async_copy(
                    idxt_hbm.at[refill_l, pl.ds(b0, _BW)], idxu, psem).start()
                pltpu.make_async_copy(
                    tgtt_hbm.at[refill_l, pl.ds(b0, _BW)], tgtu, psem).start()
            if not first:
                swait(l, obuf, ssem)

            def tr_body(vt):
                lo, hi = [], []
                for vs in range(8):
                    vcol = jnp.full((_LANES,), 0, jnp.int32) + (vt * 8 + vs)
                    lo.append(plsc.load_gather(gbuf, [lane, vcol]))
                    hi.append(plsc.load_gather(gbuf, [lane + _LANES, vcol]))
                for vs in range(8):
                    obuf[vt, vs, pl.ds(0, _LANES)] = lo[vs]
                    obuf[vt, vs, pl.ds(_LANES, _LANES)] = hi[vs]

            plsc.parallel_loop(0, vt_n, step=1, unroll=2)(tr_body)
            if refill_l is not None:
                pltpu.make_async_copy(
                    idxt_hbm.at[0, pl.ds(b0, _BW)], idxu, psem).wait()
                pltpu.make_async_copy(
                    tgtt_hbm.at[0, pl.ds(b0, _BW)], tgtu, psem).wait()
                gstart(idxu, gbuf, gsem)
            dst = y5_hbm.at[l, :, bt, :, pl.ds(lane0, _BW)]
            pltpu.make_async_copy(obuf, dst, ssem).start()

        prep(0, idxu_a, tgtu_a)
        gstart(idxu_a, gbuf_a, gsem_a)
        prep(1, idxu_b, tgtu_b)
        gstart(idxu_b, gbuf_b, gsem_b)

        unit(0, 2, idxu_a, tgtu_a, gbuf_a, obuf_a, gsem_a, ssem_a, psem_a,
             True)
        unit(1, 3, idxu_b, tgtu_b, gbuf_b, obuf_b, gsem_b, ssem_b, psem_b,
             True)

        def pair_body(p, carry):
            l = 2 * p
            unit(l, l + 2, idxu_a, tgtu_a, gbuf_a, obuf_a, gsem_a, ssem_a,
                 psem_a, False)
            unit(l + 1, l + 3, idxu_b, tgtu_b, gbuf_b, obuf_b, gsem_b,
                 ssem_b, psem_b, False)
            return carry

        lax.fori_loop(1, n_l // 2 - 1, pair_body, 0)

        unit(n_l - 2, None, idxu_a, tgtu_a, gbuf_a, obuf_a, gsem_a, ssem_a,
             psem_a, False)
        unit(n_l - 1, None, idxu_b, tgtu_b, gbuf_b, obuf_b, gsem_b, ssem_b,
             psem_b, False)
        swait(n_l - 2, obuf_a, ssem_a)
        swait(n_l - 1, obuf_b, ssem_b)
        pltpu.sync_copy(acc_v, part_hbm.at[wid])

    return sc_kernel


def kernel(idx, targets, table):
    b, l = idx.shape
    n_tok = b * l
    idxt = idx.T.astype(jnp.int32)
    tgtt = targets.T.astype(jnp.int32)
    lse = _row_lse(table)
    y5, partials = _make_sc_kernel(b, l)(table, idxt, tgtt, lse)
    logits = jnp.transpose(y5, (2, 4, 0, 1, 3)).reshape(b, l, _VOCAB)
    loss = jnp.sum(partials) / n_tok
    return (logits, loss)
